# Initial kernel scaffold; baseline (speedup 1.0000x reference)
import jax
import jax.numpy as jnp
from jax.experimental import pallas as pl


def kernel(mask, hor_vals, ver_vals, X, W1, W2, hor_idx, ver_idx, node_idx):
    raise NotImplementedError("write your pallas kernel here")


# speedup vs baseline: 3.7090x; 3.7090x over previous
import functools

import jax
import jax.numpy as jnp
from jax import lax
from jax.experimental import pallas as pl
from jax.experimental.pallas import tpu as pltpu
from jax.experimental.pallas import tpu_sc as plsc

N = 8192
E = 8192
D = 128
C = 8

NC = 2
NS = 16
NW = NC * NS
EPT = E // NW
RPT = N // NS
SR = N // 128

_HIGH = lax.Precision.HIGHEST


def _sc_body(mask_h, hv_h, vv_h, x_h, src2d_h, dst2d_h, ni_h,
             mh_out, mv_out, z_out, s_out,
             m_v, hv_v, vv_v, mh_v, mv_v, ni_v, src_v, dst_v, sidx_v,
             s2d_v, xbuf_v, z_sh, s_sh, sem):
    core = lax.axis_index("c")
    sub = lax.axis_index("s")
    wid = sub * NC + core
    base = wid * EPT
    zero16 = jnp.zeros((16,), jnp.float32)
    lane = lax.iota(jnp.int32, 16)

    pltpu.sync_copy(mask_h.at[pl.ds(base, EPT)], m_v)
    pltpu.sync_copy(hv_h.at[pl.ds(base, EPT)], hv_v)
    pltpu.sync_copy(vv_h.at[pl.ds(base, EPT)], vv_v)
    pltpu.sync_copy(src2d_h.at[pl.ds(wid * 2, 2)], src_v)
    pltpu.sync_copy(dst2d_h.at[pl.ds(wid * 2, 2)], dst_v)
    pltpu.sync_copy(ni_h, ni_v)
    ni_vec = ni_v[...]

    def _zs(r, _):
        for c in range(8):
            s2d_v[r, pl.ds(c * 16, 16)] = zero16
        return 0

    lax.fori_loop(0, SR, _zs, 0)

    @pl.when(sub == 0)
    def _():
        pltpu.sync_copy(s2d_v, s_sh)

    for g in range(EPT // 16):
        sl = pl.ds(g * 16, 16)
        sig = 1.0 / (1.0 + jnp.exp(-m_v[sl]))
        mh = hv_v[sl] * sig
        mv = vv_v[sl] * sig
        mh_v[sl] = mh
        mv_v[sl] = mv
        srcg = src_v[g // 8, pl.ds((g % 8) * 16, 16)]
        dstg = dst_v[g // 8, pl.ds((g % 8) * 16, 16)]
        w = jnp.where(srcg == ni_vec, mv, 0.0)
        plsc.addupdate_scatter(
            s2d_v, [lax.shift_right_logical(dstg, 7),
                    lax.bitwise_and(dstg, 127)], w)
    pltpu.sync_copy(mh_v, mh_out.at[pl.ds(base, EPT)])
    pltpu.sync_copy(mv_v, mv_out.at[pl.ds(base, EPT)])

    def _zx(r, _):
        for c in range(D // 16):
            xbuf_v[r, pl.ds(c * 16, 16)] = zero16
        return 0

    lax.fori_loop(0, EPT, _zx, 0)
    rbase = sub * RPT
    pltpu.sync_copy(xbuf_v, z_sh.at[pl.ds(rbase, EPT)])
    pltpu.sync_copy(xbuf_v, z_sh.at[pl.ds(rbase + EPT, EPT)])
    plsc.subcore_barrier()

    for j in range(2):
        pltpu.async_copy(x_h.at[src_v.at[j]],
                         xbuf_v.at[pl.ds(j * 128, 128)], sem).wait()

    def _srow(r, _):
        idx16 = jnp.full((16,), r, jnp.int32)
        sc = plsc.load_gather(mh_v, [idx16])
        for c in range(D // 16):
            xbuf_v[r, pl.ds(c * 16, 16)] = xbuf_v[r, pl.ds(c * 16, 16)] * sc
        return 0

    lax.fori_loop(0, EPT, _srow, 0)

    for j in range(2):
        pltpu.sync_copy(xbuf_v.at[pl.ds(j * 128, 128)],
                        z_sh.at[dst_v.at[j]], add=True)
    for g in range(SR // 16):
        sidx_v[pl.ds(g * 16, 16)] = lane + (g * 16)
    pltpu.sync_copy(s2d_v, s_sh.at[sidx_v], add=True)
    plsc.subcore_barrier()

    pltpu.sync_copy(z_sh.at[pl.ds(rbase, RPT)],
                    z_out.at[core].at[pl.ds(rbase, RPT)])
    pltpu.sync_copy(s_sh.at[pl.ds(sub * 4, 4)],
                    s_out.at[core].at[pl.ds(sub * 4, 4)])


@functools.partial(
    pl.kernel,
    mesh=plsc.VectorSubcoreMesh(core_axis_name="c", subcore_axis_name="s",
                                num_cores=NC, num_subcores=NS),
    out_type=[
        jax.ShapeDtypeStruct((E,), jnp.float32),
        jax.ShapeDtypeStruct((E,), jnp.float32),
        jax.ShapeDtypeStruct((NC, N, D), jnp.float32),
        jax.ShapeDtypeStruct((NC, SR, 128), jnp.float32),
    ],
    scratch_types=[
        pltpu.VMEM((EPT,), jnp.float32),
        pltpu.VMEM((EPT,), jnp.float32),
        pltpu.VMEM((EPT,), jnp.float32),
        pltpu.VMEM((EPT,), jnp.float32),
        pltpu.VMEM((EPT,), jnp.float32),
        pltpu.VMEM((16,), jnp.int32),
        pltpu.VMEM((2, 128), jnp.int32),
        pltpu.VMEM((2, 128), jnp.int32),
        pltpu.VMEM((SR,), jnp.int32),
        pltpu.VMEM((SR, 128), jnp.float32),
        pltpu.VMEM((EPT, D), jnp.float32),
        pltpu.VMEM_SHARED((N, D), jnp.float32),
        pltpu.VMEM_SHARED((SR, 128), jnp.float32),
        pltpu.SemaphoreType.DMA,
    ],
    compiler_params=pltpu.CompilerParams(needs_layout_passes=False),
)
def _sc_spmm(*refs):
    _sc_body(*refs)


def _tc_body(z_ref, s_ref, w1_ref, w2_ref, o_ref, acc_ref):
    i = pl.program_id(0)
    z = z_ref[0] + z_ref[1]
    s2 = s_ref[0] + s_ref[1]
    h = jnp.maximum(lax.dot(z, w1_ref[...], precision=_HIGH), 0.0)
    r = lax.dot(s2[0:1, :], h[0:128, :], precision=_HIGH)
    for a in range(1, _ROWS_BLK // 128):
        r += lax.dot(s2[a:a + 1, :], h[a * 128:(a + 1) * 128, :],
                     precision=_HIGH)

    @pl.when(i == 0)
    def _():
        acc_ref[...] = jnp.zeros_like(acc_ref)

    acc_ref[...] += r

    @pl.when(i == pl.num_programs(0) - 1)
    def _():
        y = lax.dot(acc_ref[...], w2_ref[...], precision=_HIGH)
        lanes = lax.broadcasted_iota(jnp.int32, (1, 128), 1)
        ym = jnp.where(lanes < C, y, -1e30)
        e = jnp.where(lanes < C, jnp.exp(ym - jnp.max(ym)), 0.0)
        o_ref[...] = e / jnp.sum(e)


_ROWS_BLK = 1024

_tc_finish = pl.pallas_call(
    _tc_body,
    grid=(N // _ROWS_BLK,),
    in_specs=[
        pl.BlockSpec((NC, _ROWS_BLK, D), lambda i: (0, i, 0)),
        pl.BlockSpec((NC, _ROWS_BLK // 128, 128), lambda i: (0, i, 0)),
        pl.BlockSpec((D, D), lambda i: (0, 0)),
        pl.BlockSpec((D, 128), lambda i: (0, 0)),
    ],
    out_specs=pl.BlockSpec((1, 128), lambda i: (0, 0)),
    out_shape=jax.ShapeDtypeStruct((1, 128), jnp.float32),
    scratch_shapes=[pltpu.VMEM((1, 128), jnp.float32)],
)


def kernel(mask, hor_vals, ver_vals, X, W1, W2, hor_idx, ver_idx, node_idx):
    src2d = hor_idx[1].reshape(E // 128, 128)
    dst2d = hor_idx[0].reshape(E // 128, 128)
    ni16 = jnp.full((16,), node_idx, jnp.int32)
    mh, mv, z, s = _sc_spmm(mask, hor_vals, ver_vals, X, src2d, dst2d, ni16)
    w2pad = jnp.pad(W2, ((0, 0), (0, 128 - C)))
    res = _tc_finish(z, s, W1, w2pad)[0, :C]
    return (res, mh, mv)
